# baseline (speedup 1.0000x reference)
import jax
import jax.numpy as jnp
from jax import lax
from jax.experimental import pallas as pl
from jax.experimental.pallas import tpu as pltpu
from jax.experimental.pallas import tpu_sc as plsc

NUM_CORES = 2
NUM_SUBCORES = 16
NUM_WORKERS = NUM_CORES * NUM_SUBCORES
LANES = 16
BATCH = 16384
DIM = 64
B_PER_W = BATCH // NUM_WORKERS
CHUNK = 128
NCHUNK = B_PER_W // CHUNK
GROUPS = CHUNK // LANES


def _gmf_body(uidx_hbm, iidx_hbm, utab_hbm, itab_hbm, w_hbm, b_hbm,
              out_hbm,
              idx_u_v, idx_i_v, rows_u_v, rows_i_v,
              w_v, b_v, tr_v, out_v,
              sem0, sem1, sem2, sem3):
    sems = [sem0, sem1, sem2, sem3]
    wid = lax.axis_index("s") * NUM_CORES + lax.axis_index("c")
    base = wid * B_PER_W

    pltpu.sync_copy(w_hbm, w_v)
    pltpu.sync_copy(b_hbm, b_v)

    for k in range(NCHUNK):
        pltpu.sync_copy(uidx_hbm.at[pl.ds(base + k * CHUNK, CHUNK)],
                        idx_u_v.at[k])
        pltpu.sync_copy(iidx_hbm.at[pl.ds(base + k * CHUNK, CHUNK)],
                        idx_i_v.at[k])

    handles = []
    for k in range(NCHUNK):
        hu = pltpu.async_copy(utab_hbm.at[idx_u_v.at[k]], rows_u_v.at[k],
                              sems[k])
        hi = pltpu.async_copy(itab_hbm.at[idx_i_v.at[k]], rows_i_v.at[k],
                              sems[k])
        handles.append((hu, hi))

    wv = [w_v[pl.ds(c * LANES, LANES)] for c in range(4)]
    bias = b_v[...]
    lane_x16 = lax.iota(jnp.int32, LANES) * LANES

    for k in range(NCHUNK):
        hu, hi = handles[k]
        hu.wait()
        hi.wait()
        uk = rows_u_v.at[k]
        ik = rows_i_v.at[k]

        def group_body(g, carry, uk=uk, ik=ik, k=k):
            for bi in range(LANES):
                b = g * LANES + bi
                s = None
                for c in range(4):
                    t = (uk[b, pl.ds(c * LANES, LANES)]
                         * ik[b, pl.ds(c * LANES, LANES)]
                         * wv[c])
                    s = t if s is None else s + t
                tr_v[pl.ds(bi * LANES, LANES)] = s
            acc = bias
            for d in range(LANES):
                col = plsc.load_gather(tr_v, [lane_x16 + d])
                acc = acc + col
            out16 = 1.0 / (1.0 + jnp.exp(-acc))
            out_v[pl.ds(k * CHUNK + g * LANES, LANES)] = out16
            return carry

        lax.fori_loop(0, GROUPS, group_body, 0)

    pltpu.sync_copy(out_v, out_hbm.at[pl.ds(base, B_PER_W)])


def _build():
    mesh = plsc.VectorSubcoreMesh(core_axis_name="c", subcore_axis_name="s")
    return pl.kernel(
        _gmf_body,
        mesh=mesh,
        compiler_params=pltpu.CompilerParams(
            needs_layout_passes=False, use_tc_tiling_on_sc=False),
        out_type=jax.ShapeDtypeStruct((BATCH,), jnp.float32),
        scratch_types=[
            pltpu.VMEM((NCHUNK, CHUNK), jnp.int32),
            pltpu.VMEM((NCHUNK, CHUNK), jnp.int32),
            pltpu.VMEM((NCHUNK, CHUNK, DIM), jnp.float32),
            pltpu.VMEM((NCHUNK, CHUNK, DIM), jnp.float32),
            pltpu.VMEM((DIM,), jnp.float32),
            pltpu.VMEM((LANES,), jnp.float32),
            pltpu.VMEM((LANES * LANES,), jnp.float32),
            pltpu.VMEM((B_PER_W,), jnp.float32),
            pltpu.SemaphoreType.DMA,
            pltpu.SemaphoreType.DMA,
            pltpu.SemaphoreType.DMA,
            pltpu.SemaphoreType.DMA,
        ],
    )


def kernel(user_indices, item_indices, user_table, item_table, W, b):
    uidx = user_indices.astype(jnp.int32)
    iidx = item_indices.astype(jnp.int32)
    w_flat = W.reshape(DIM).astype(jnp.float32)
    b16 = jnp.broadcast_to(b.astype(jnp.float32), (LANES,))
    out = _build()(uidx, iidx, user_table, item_table, w_flat, b16)
    return out.reshape(BATCH, 1)
